# Initial kernel scaffold; baseline (speedup 1.0000x reference)
import jax
import jax.numpy as jnp
from jax.experimental import pallas as pl


def kernel(memory, write_weights, erase_vector, write_vector, prev_link, precedence_weights, free_gate, read_weights, prev_usage):
    raise NotImplementedError("write your pallas kernel here")


# speedup vs baseline: 1.9050x; 1.9050x over previous
import functools

import jax
import jax.numpy as jnp
from jax import lax
from jax.experimental import pallas as pl
from jax.experimental.pallas import tpu as pltpu
from jax.experimental.pallas import tpu_sc as plsc

B = 64
N = 512
W = 64
NR = 4

NUM_WORKERS = 32
BATCH_PER_WORKER = B // NUM_WORKERS
ROWS = 64
CHUNKS = N // ROWS
LANES = 16
NBLK = N // LANES


def _link_body(ww_hbm, pp_hbm, prev_hbm, out_hbm, ww_v, pp_v, buf_v, sem):
    wid = lax.axis_index("s") * 2 + lax.axis_index("c")

    def do_batch(bi, _):
        b = wid * BATCH_PER_WORKER + bi
        pltpu.sync_copy(ww_hbm.at[b], ww_v)
        pltpu.sync_copy(pp_hbm.at[b], pp_v)

        def do_chunk(ck, _):
            row0 = ck * ROWS
            base = b * N + row0
            pltpu.sync_copy(prev_hbm.at[pl.ds(base, ROWS)], buf_v)

            def do_rowgroup(rg, _):
                r0 = rg * LANES
                wwi_vec = ww_v[pl.ds(row0 + r0, LANES)]
                om_vec = 1.0 - wwi_vec
                wwis = [wwi_vec[l] for l in range(LANES)]
                oms = [om_vec[l] for l in range(LANES)]

                def do_jb(jb, _):
                    sl = pl.ds(jb * LANES, LANES)
                    a_blk = ww_v[sl]
                    c_blk = pp_v[sl]
                    for l in range(LANES):
                        s = oms[l] - a_blk
                        o = s * buf_v[r0 + l, sl] + wwis[l] * c_blk
                        buf_v[r0 + l, sl] = o
                    return 0

                lax.fori_loop(0, NBLK, do_jb, 0)

                dsl = pl.ds(row0 + r0, LANES)
                lane = lax.iota(jnp.int32, LANES)
                for l in range(LANES):
                    v = buf_v[r0 + l, dsl]
                    buf_v[r0 + l, dsl] = jnp.where(lane == l, 0.0, v)
                return 0

            lax.fori_loop(0, ROWS // LANES, do_rowgroup, 0)

            pltpu.sync_copy(buf_v, out_hbm.at[pl.ds(base, ROWS)])
            return 0

        lax.fori_loop(0, CHUNKS, do_chunk, 0, unroll=False)
        return 0

    lax.fori_loop(0, BATCH_PER_WORKER, do_batch, 0, unroll=False)


@functools.partial(jax.jit, static_argnums=())
def _link_update(ww, pp, prev_link):
    kern = pl.kernel(
        _link_body,
        out_type=jax.ShapeDtypeStruct((B * N, N), jnp.float32),
        mesh=plsc.VectorSubcoreMesh(core_axis_name="c", subcore_axis_name="s"),
        scratch_types=[
            pltpu.VMEM((N,), jnp.float32),
            pltpu.VMEM((N,), jnp.float32),
            pltpu.VMEM((ROWS, N), jnp.float32),
            pltpu.SemaphoreType.DMA,
        ],
    )
    return kern(ww, pp, prev_link)


def _tc_body(mem_ref, ww_ref, ev_ref, wv_ref, pw_ref, fg_ref, rw_ref, pu_ref,
             mem_out_ref, prec_ref, usage_ref, read_ref):
    mem = mem_ref[...]
    ww = ww_ref[...]
    ev = ev_ref[...]
    wv = wv_ref[...]
    pw = pw_ref[...]
    fg = fg_ref[...]
    rw = rw_ref[...]
    pu = pu_ref[...]

    keep = 1.0 - ww[:, :, None] * ev[:, None, :]
    add = ww[:, :, None] * wv[:, None, :]
    mem_out = mem * keep + add
    mem_out_ref[...] = mem_out

    wsum = jnp.sum(ww, axis=1, keepdims=True)
    prec_ref[...] = (1.0 - wsum) * pw + ww

    u1 = pu + (1.0 - pu) * ww
    phi = 1.0
    for r in range(NR):
        phi = phi * (1.0 - fg[:, r][:, None] * rw[:, r, :])
    usage_ref[...] = u1 * phi

    gb = mem.shape[0]
    for g in range(gb):
        read_ref[g, :, :] = jnp.dot(rw[g], mem_out[g],
                                    preferred_element_type=jnp.float32)


def _tc_update(memory, ww, ev, wv, pw, fg, rw, pu):
    GB = 8
    grid = (B // GB,)
    bspec = lambda shape: pl.BlockSpec((GB,) + shape,
                                       lambda i: (i,) + (0,) * len(shape))
    return pl.pallas_call(
        _tc_body,
        grid=grid,
        in_specs=[
            bspec((N, W)),
            bspec((N,)),
            bspec((W,)),
            bspec((W,)),
            bspec((N,)),
            bspec((NR,)),
            bspec((NR, N)),
            bspec((N,)),
        ],
        out_specs=[
            bspec((N, W)),
            bspec((N,)),
            bspec((N,)),
            bspec((NR, W)),
        ],
        out_shape=[
            jax.ShapeDtypeStruct((B, N, W), jnp.float32),
            jax.ShapeDtypeStruct((B, N), jnp.float32),
            jax.ShapeDtypeStruct((B, N), jnp.float32),
            jax.ShapeDtypeStruct((B, NR, W), jnp.float32),
        ],
    )(memory, ww, ev, wv, pw, fg, rw, pu)


def kernel(memory, write_weights, erase_vector, write_vector, prev_link,
           precedence_weights, free_gate, read_weights, prev_usage):
    ww = write_weights.reshape(B, N)
    ev = erase_vector.reshape(B, W)
    wv = write_vector.reshape(B, W)
    pw = precedence_weights.reshape(B, N)

    link_flat = _link_update(ww, pw, prev_link.reshape(B * N, N))
    link = link_flat.reshape(B, 1, N, N)

    mem_out, prec, usage, read_words = _tc_update(
        memory, ww, ev, wv, pw, free_gate, read_weights, prev_usage)

    return (mem_out, link.reshape(prev_link.shape),
            prec.reshape(precedence_weights.shape), usage, read_words)


# speedup vs baseline: 2.0107x; 1.0555x over previous
import functools

import jax
import jax.numpy as jnp
from jax import lax
from jax.experimental import pallas as pl
from jax.experimental.pallas import tpu as pltpu
from jax.experimental.pallas import tpu_sc as plsc

B = 64
N = 512
W = 64
NR = 4

NUM_WORKERS = 32
BATCH_PER_WORKER = B // NUM_WORKERS
ROWS = 32
CHUNKS = N // ROWS
TOTAL = BATCH_PER_WORKER * CHUNKS
LANES = 16
NBLK = N // LANES


def _link_body(ww_hbm, pp_hbm, prev_hbm, out_hbm,
               ww_v, pp_v, in0, in1, out0, out1, si0, si1, so0, so1):
    wid = lax.axis_index("s") * 2 + lax.axis_index("c")
    in_bufs, out_bufs = (in0, in1), (out0, out1)
    in_sems, out_sems = (si0, si1), (so0, so1)

    for bi in range(BATCH_PER_WORKER):
        b = wid * BATCH_PER_WORKER + bi
        pltpu.sync_copy(ww_hbm.at[b], ww_v.at[bi])
        pltpu.sync_copy(pp_hbm.at[b], pp_v.at[bi])

    def hbm_base(c):
        bi = c // CHUNKS
        row0 = (c % CHUNKS) * ROWS
        b = wid * BATCH_PER_WORKER + bi
        return b * N + row0

    def in_copy(c, s):
        return pltpu.make_async_copy(
            prev_hbm.at[pl.ds(hbm_base(c), ROWS)], in_bufs[s], in_sems[s])

    def out_copy(c, s):
        return pltpu.make_async_copy(
            out_bufs[s], out_hbm.at[pl.ds(hbm_base(c), ROWS)], out_sems[s])

    def compute(c, s):
        bi = c // CHUNKS
        row0 = (c % CHUNKS) * ROWS
        src, dst = in_bufs[s], out_bufs[s]

        def do_rowgroup(rg, _):
            r0 = rg * LANES
            wwi_vec = ww_v[bi, pl.ds(row0 + r0, LANES)]
            om_vec = 1.0 - wwi_vec
            wwis = [wwi_vec[l] for l in range(LANES)]
            oms = [om_vec[l] for l in range(LANES)]

            def do_jb(jb, _):
                sl = pl.ds(jb * LANES, LANES)
                a_blk = ww_v[bi, sl]
                c_blk = pp_v[bi, sl]
                for l in range(LANES):
                    s_ = oms[l] - a_blk
                    dst[r0 + l, sl] = s_ * src[r0 + l, sl] + wwis[l] * c_blk
                return 0

            lax.fori_loop(0, NBLK, do_jb, 0)

            dsl = pl.ds(row0 + r0, LANES)
            lane = lax.iota(jnp.int32, LANES)
            for l in range(LANES):
                v = dst[r0 + l, dsl]
                dst[r0 + l, dsl] = jnp.where(lane == l, 0.0, v)
            return 0

        lax.fori_loop(0, ROWS // LANES, do_rowgroup, 0)

    in_copy(0, 0).start()
    in_copy(1, 1).start()

    def step(g, _):
        for s in range(2):
            c = 2 * g + s
            in_copy(c, s).wait()
            @pl.when(c >= 2)
            def _():
                out_copy(c - 2, s).wait()
            compute(c, s)
            out_copy(c, s).start()

            @pl.when(c + 2 < TOTAL)
            def _():
                in_copy(c + 2, s).start()
        return 0

    lax.fori_loop(0, TOTAL // 2, step, 0)
    out_copy(TOTAL - 2, 0).wait()
    out_copy(TOTAL - 1, 1).wait()


@functools.partial(jax.jit, static_argnums=())
def _link_update(ww, pp, prev_link):
    kern = pl.kernel(
        _link_body,
        out_type=jax.ShapeDtypeStruct((B * N, N), jnp.float32),
        mesh=plsc.VectorSubcoreMesh(core_axis_name="c", subcore_axis_name="s"),
        scratch_types=[
            pltpu.VMEM((BATCH_PER_WORKER, N), jnp.float32),
            pltpu.VMEM((BATCH_PER_WORKER, N), jnp.float32),
            pltpu.VMEM((ROWS, N), jnp.float32),
            pltpu.VMEM((ROWS, N), jnp.float32),
            pltpu.VMEM((ROWS, N), jnp.float32),
            pltpu.VMEM((ROWS, N), jnp.float32),
            pltpu.SemaphoreType.DMA,
            pltpu.SemaphoreType.DMA,
            pltpu.SemaphoreType.DMA,
            pltpu.SemaphoreType.DMA,
        ],
    )
    return kern(ww, pp, prev_link)


def _tc_body(mem_ref, ww_ref, ev_ref, wv_ref, pw_ref, fg_ref, rw_ref, pu_ref,
             mem_out_ref, prec_ref, usage_ref, read_ref):
    mem = mem_ref[...]
    ww = ww_ref[...]
    ev = ev_ref[...]
    wv = wv_ref[...]
    pw = pw_ref[...]
    fg = fg_ref[...]
    rw = rw_ref[...]
    pu = pu_ref[...]

    keep = 1.0 - ww[:, :, None] * ev[:, None, :]
    add = ww[:, :, None] * wv[:, None, :]
    mem_out = mem * keep + add
    mem_out_ref[...] = mem_out

    wsum = jnp.sum(ww, axis=1, keepdims=True)
    prec_ref[...] = (1.0 - wsum) * pw + ww

    u1 = pu + (1.0 - pu) * ww
    phi = 1.0
    for r in range(NR):
        phi = phi * (1.0 - fg[:, r][:, None] * rw[:, r, :])
    usage_ref[...] = u1 * phi

    gb = mem.shape[0]
    for g in range(gb):
        read_ref[g, :, :] = jnp.dot(rw[g], mem_out[g],
                                    preferred_element_type=jnp.float32)


def _tc_update(memory, ww, ev, wv, pw, fg, rw, pu):
    GB = 8
    grid = (B // GB,)
    bspec = lambda shape: pl.BlockSpec((GB,) + shape,
                                       lambda i: (i,) + (0,) * len(shape))
    return pl.pallas_call(
        _tc_body,
        grid=grid,
        in_specs=[
            bspec((N, W)),
            bspec((N,)),
            bspec((W,)),
            bspec((W,)),
            bspec((N,)),
            bspec((NR,)),
            bspec((NR, N)),
            bspec((N,)),
        ],
        out_specs=[
            bspec((N, W)),
            bspec((N,)),
            bspec((N,)),
            bspec((NR, W)),
        ],
        out_shape=[
            jax.ShapeDtypeStruct((B, N, W), jnp.float32),
            jax.ShapeDtypeStruct((B, N), jnp.float32),
            jax.ShapeDtypeStruct((B, N), jnp.float32),
            jax.ShapeDtypeStruct((B, NR, W), jnp.float32),
        ],
    )(memory, ww, ev, wv, pw, fg, rw, pu)


def kernel(memory, write_weights, erase_vector, write_vector, prev_link,
           precedence_weights, free_gate, read_weights, prev_usage):
    ww = write_weights.reshape(B, N)
    ev = erase_vector.reshape(B, W)
    wv = write_vector.reshape(B, W)
    pw = precedence_weights.reshape(B, N)

    link_flat = _link_update(ww, pw, prev_link.reshape(B * N, N))
    link = link_flat.reshape(B, 1, N, N)

    mem_out, prec, usage, read_words = _tc_update(
        memory, ww, ev, wv, pw, free_gate, read_weights, prev_usage)

    return (mem_out, link.reshape(prev_link.shape),
            prec.reshape(precedence_weights.shape), usage, read_words)


# speedup vs baseline: 2.8677x; 1.4262x over previous
import functools

import jax
import jax.numpy as jnp
from jax import lax
from jax.experimental import pallas as pl
from jax.experimental.pallas import tpu as pltpu
from jax.experimental.pallas import tpu_sc as plsc

B = 64
N = 512
W = 64
NR = 4

NUM_WORKERS = 32
BATCH_PER_WORKER = B // NUM_WORKERS
ROWS = 32
CHUNKS = N // ROWS
TOTAL = BATCH_PER_WORKER * CHUNKS
LANES = 16
NBLK = N // LANES


def _link_body(ww_hbm, pp_hbm, prev_hbm, out_hbm,
               ww_v, pp_v, in0, in1, out0, out1, si0, si1, so0, so1):
    wid = lax.axis_index("s") * 2 + lax.axis_index("c")
    in_bufs, out_bufs = (in0, in1), (out0, out1)
    in_sems, out_sems = (si0, si1), (so0, so1)

    for bi in range(BATCH_PER_WORKER):
        b = wid * BATCH_PER_WORKER + bi
        pltpu.sync_copy(ww_hbm.at[b], ww_v.at[bi])
        pltpu.sync_copy(pp_hbm.at[b], pp_v.at[bi])

    def hbm_base(c):
        bi = c // CHUNKS
        row0 = (c % CHUNKS) * ROWS
        b = wid * BATCH_PER_WORKER + bi
        return b * N + row0

    def in_copy(c, s):
        return pltpu.make_async_copy(
            prev_hbm.at[pl.ds(hbm_base(c), ROWS)], in_bufs[s], in_sems[s])

    def out_copy(c, s):
        return pltpu.make_async_copy(
            out_bufs[s], out_hbm.at[pl.ds(hbm_base(c), ROWS)], out_sems[s])

    def compute(c, s):
        bi = c // CHUNKS
        row0 = (c % CHUNKS) * ROWS
        src, dst = in_bufs[s], out_bufs[s]

        @plsc.parallel_loop(0, ROWS // LANES, step=1)
        def do_rowgroup(rg):
            r0 = rg * LANES
            wwi_vec = ww_v[bi, pl.ds(row0 + r0, LANES)]
            om_vec = 1.0 - wwi_vec
            wwis = [wwi_vec[l] for l in range(LANES)]
            oms = [om_vec[l] for l in range(LANES)]

            @plsc.parallel_loop(0, NBLK, step=1)
            def do_jb(jb):
                sl = pl.ds(jb * LANES, LANES)
                a_blk = ww_v[bi, sl]
                c_blk = pp_v[bi, sl]
                for l in range(LANES):
                    s_ = oms[l] - a_blk
                    dst[r0 + l, sl] = s_ * src[r0 + l, sl] + wwis[l] * c_blk

            dsl = pl.ds(row0 + r0, LANES)
            lane = lax.iota(jnp.int32, LANES)
            for l in range(LANES):
                v = dst[r0 + l, dsl]
                dst[r0 + l, dsl] = jnp.where(lane == l, 0.0, v)

    in_copy(0, 0).start()
    in_copy(1, 1).start()

    def step(g, _):
        for s in range(2):
            c = 2 * g + s
            in_copy(c, s).wait()
            @pl.when(c >= 2)
            def _():
                out_copy(c - 2, s).wait()
            compute(c, s)
            out_copy(c, s).start()

            @pl.when(c + 2 < TOTAL)
            def _():
                in_copy(c + 2, s).start()
        return 0

    lax.fori_loop(0, TOTAL // 2, step, 0)
    out_copy(TOTAL - 2, 0).wait()
    out_copy(TOTAL - 1, 1).wait()


@functools.partial(jax.jit, static_argnums=())
def _link_update(ww, pp, prev_link):
    kern = pl.kernel(
        _link_body,
        out_type=jax.ShapeDtypeStruct((B * N, N), jnp.float32),
        mesh=plsc.VectorSubcoreMesh(core_axis_name="c", subcore_axis_name="s"),
        scratch_types=[
            pltpu.VMEM((BATCH_PER_WORKER, N), jnp.float32),
            pltpu.VMEM((BATCH_PER_WORKER, N), jnp.float32),
            pltpu.VMEM((ROWS, N), jnp.float32),
            pltpu.VMEM((ROWS, N), jnp.float32),
            pltpu.VMEM((ROWS, N), jnp.float32),
            pltpu.VMEM((ROWS, N), jnp.float32),
            pltpu.SemaphoreType.DMA,
            pltpu.SemaphoreType.DMA,
            pltpu.SemaphoreType.DMA,
            pltpu.SemaphoreType.DMA,
        ],
    )
    return kern(ww, pp, prev_link)


def _tc_body(mem_ref, ww_ref, ev_ref, wv_ref, pw_ref, fg_ref, rw_ref, pu_ref,
             mem_out_ref, prec_ref, usage_ref, read_ref):
    mem = mem_ref[...]
    ww = ww_ref[...]
    ev = ev_ref[...]
    wv = wv_ref[...]
    pw = pw_ref[...]
    fg = fg_ref[...]
    rw = rw_ref[...]
    pu = pu_ref[...]

    keep = 1.0 - ww[:, :, None] * ev[:, None, :]
    add = ww[:, :, None] * wv[:, None, :]
    mem_out = mem * keep + add
    mem_out_ref[...] = mem_out

    wsum = jnp.sum(ww, axis=1, keepdims=True)
    prec_ref[...] = (1.0 - wsum) * pw + ww

    u1 = pu + (1.0 - pu) * ww
    phi = 1.0
    for r in range(NR):
        phi = phi * (1.0 - fg[:, r][:, None] * rw[:, r, :])
    usage_ref[...] = u1 * phi

    gb = mem.shape[0]
    for g in range(gb):
        read_ref[g, :, :] = jnp.dot(rw[g], mem_out[g],
                                    preferred_element_type=jnp.float32)


def _tc_update(memory, ww, ev, wv, pw, fg, rw, pu):
    GB = 8
    grid = (B // GB,)
    bspec = lambda shape: pl.BlockSpec((GB,) + shape,
                                       lambda i: (i,) + (0,) * len(shape))
    return pl.pallas_call(
        _tc_body,
        grid=grid,
        in_specs=[
            bspec((N, W)),
            bspec((N,)),
            bspec((W,)),
            bspec((W,)),
            bspec((N,)),
            bspec((NR,)),
            bspec((NR, N)),
            bspec((N,)),
        ],
        out_specs=[
            bspec((N, W)),
            bspec((N,)),
            bspec((N,)),
            bspec((NR, W)),
        ],
        out_shape=[
            jax.ShapeDtypeStruct((B, N, W), jnp.float32),
            jax.ShapeDtypeStruct((B, N), jnp.float32),
            jax.ShapeDtypeStruct((B, N), jnp.float32),
            jax.ShapeDtypeStruct((B, NR, W), jnp.float32),
        ],
    )(memory, ww, ev, wv, pw, fg, rw, pu)


def kernel(memory, write_weights, erase_vector, write_vector, prev_link,
           precedence_weights, free_gate, read_weights, prev_usage):
    ww = write_weights.reshape(B, N)
    ev = erase_vector.reshape(B, W)
    wv = write_vector.reshape(B, W)
    pw = precedence_weights.reshape(B, N)

    link_flat = _link_update(ww, pw, prev_link.reshape(B * N, N))
    link = link_flat.reshape(B, 1, N, N)

    mem_out, prec, usage, read_words = _tc_update(
        memory, ww, ev, wv, pw, free_gate, read_weights, prev_usage)

    return (mem_out, link.reshape(prev_link.shape),
            prec.reshape(precedence_weights.shape), usage, read_words)
